# baseline (speedup 1.0000x reference)
import jax
import jax.numpy as jnp
from jax import lax
from jax.experimental import pallas as pl
from jax.experimental.pallas import tpu as pltpu

_N_LAYERS = 20
_HALF = _N_LAYERS // 2
_LPS = 2
_N_CHAIN = _HALF // _LPS
_MERGE = _N_CHAIN
_APPLY0 = _MERGE + 1
_TOKENS = 4096
_F = 1024
_AF = _F + 8
_APPLY_TILE = 512
_N_APPLY = _TOKENS // _APPLY_TILE

_NT = (((1,), (1,)), ((), ()))
_NN = (((1,), (0,)), ((), ()))


def _bf(v):
    return v.astype(jnp.bfloat16)


def _fold(a, w, brow):
    t = lax.dot_general(_bf(a), _bf(w), _NT,
                        preferred_element_type=jnp.float32)
    return jnp.concatenate([t[0:_F, :], t[_F:, :] + brow], axis=0)


def _aug(w_ref, b_ref):
    return jnp.concatenate(
        [w_ref[0].T, jnp.broadcast_to(b_ref[0], (_AF - _F, _F))], axis=0)


def _body(x_ref, wa0_ref, wa1_ref, wb0_ref, wb1_ref,
          ba0_ref, ba1_ref, bb0_ref, bb1_ref, out_ref, a_scr, b_scr):
    i = pl.program_id(0)

    @pl.when(i == 0)
    def _init():
        a_scr[...] = _fold(_aug(wa0_ref, ba0_ref), wa1_ref[0], ba1_ref[0])
        b_scr[...] = _fold(_aug(wb0_ref, bb0_ref), wb1_ref[0], bb1_ref[0])

    @pl.when((i > 0) & (i < _N_CHAIN))
    def _chain():
        a = _fold(a_scr[...], wa0_ref[0], ba0_ref[0])
        b = _fold(b_scr[...], wb0_ref[0], bb0_ref[0])
        a_scr[...] = _fold(a, wa1_ref[0], ba1_ref[0])
        b_scr[...] = _fold(b, wb1_ref[0], bb1_ref[0])

    @pl.when(i == _MERGE)
    def _merge():
        t = lax.dot_general(
            _bf(a_scr[...]), _bf(b_scr[0:_F, :]), _NN,
            preferred_element_type=jnp.float32)
        a_scr[0:_F, :] = t[0:_F, :]
        a_scr[_F:, :] = t[_F:, :] + b_scr[_F:, :]

    @pl.when(i >= _APPLY0)
    def _apply():
        out_ref[...] = lax.dot_general(
            _bf(x_ref[...]), _bf(a_scr[0:_F, :]), _NN,
            preferred_element_type=jnp.float32) + a_scr[_F:_F + 1, :]


def kernel(x, W, b):
    b3 = b.reshape(_N_LAYERS, 1, _F)

    def _wspec(off):
        return pl.BlockSpec(
            (1, _F, _F),
            lambda i, off=off: (
                _LPS * jnp.minimum(i, _N_CHAIN - 1) + off, 0, 0))

    def _bspec(off):
        return pl.BlockSpec(
            (1, 1, _F),
            lambda i, off=off: (
                _LPS * jnp.minimum(i, _N_CHAIN - 1) + off, 0, 0))

    return pl.pallas_call(
        _body,
        grid=(_APPLY0 + _N_APPLY,),
        in_specs=[
            pl.BlockSpec((_APPLY_TILE, _F),
                         lambda i: (jnp.maximum(i - _APPLY0, 0), 0)),
            _wspec(0), _wspec(1), _wspec(_HALF), _wspec(_HALF + 1),
            _bspec(0), _bspec(1), _bspec(_HALF), _bspec(_HALF + 1),
        ],
        out_specs=pl.BlockSpec((_APPLY_TILE, _F),
                               lambda i: (jnp.maximum(i - _APPLY0, 0), 0)),
        out_shape=jax.ShapeDtypeStruct((_TOKENS, _F), jnp.float32),
        scratch_shapes=[
            pltpu.VMEM((_AF, _F), jnp.float32),
            pltpu.VMEM((_AF, _F), jnp.float32),
        ],
    )(x, W, W, W, W, b3, b3, b3, b3)


# speedup vs baseline: 1.0093x; 1.0093x over previous
import jax
import jax.numpy as jnp
from jax import lax
from jax.experimental import pallas as pl
from jax.experimental.pallas import tpu as pltpu

_N_LAYERS = 20
_LPS = 4
_N_CHAIN = _N_LAYERS // _LPS
_TOKENS = 4096
_F = 1024
_CB = 8
_APPLY_TILE = 512
_N_APPLY = _TOKENS // _APPLY_TILE

_NT = (((1,), (1,)), ((), ()))
_NN = (((1,), (0,)), ((), ()))


def _bf(v):
    return v.astype(jnp.bfloat16)


def _fold(q_bf, c_f32, w, brow):
    a_bf = jnp.concatenate([q_bf, _bf(c_f32)], axis=0)
    t = lax.dot_general(a_bf, _bf(w), _NT,
                        preferred_element_type=jnp.float32)
    return _bf(t[0:_F, :]), t[_F:, :] + brow


def _body(x_ref, w_ref, b_ref, out_ref, q_scr, c_scr):
    i = pl.program_id(0)

    @pl.when(i == 0)
    def _init():
        q = _bf(w_ref[0].T)
        c = jnp.broadcast_to(b_ref[0, 0], (_CB, _F)).astype(jnp.float32)
        for j in range(1, _LPS):
            q, c = _fold(q, c, w_ref[j], b_ref[j, 0])
        q_scr[...] = q
        c_scr[...] = c

    @pl.when((i > 0) & (i < _N_CHAIN))
    def _chain():
        q, c = q_scr[...], c_scr[...]
        for j in range(_LPS):
            q, c = _fold(q, c, w_ref[j], b_ref[j, 0])
        q_scr[...] = q
        c_scr[...] = c

    @pl.when(i >= _N_CHAIN)
    def _apply():
        out_ref[...] = lax.dot_general(
            _bf(x_ref[...]), q_scr[...], _NN,
            preferred_element_type=jnp.float32) + c_scr[0:1, :]


def kernel(x, W, b):
    return pl.pallas_call(
        _body,
        grid=(_N_CHAIN + _N_APPLY,),
        in_specs=[
            pl.BlockSpec((_APPLY_TILE, _F),
                         lambda i: (jnp.maximum(i - _N_CHAIN, 0), 0)),
            pl.BlockSpec((_LPS, _F, _F),
                         lambda i: (jnp.minimum(i, _N_CHAIN - 1), 0, 0)),
            pl.BlockSpec((_LPS, 1, _F),
                         lambda i: (jnp.minimum(i, _N_CHAIN - 1), 0, 0)),
        ],
        out_specs=pl.BlockSpec((_APPLY_TILE, _F),
                               lambda i: (jnp.maximum(i - _N_CHAIN, 0), 0)),
        out_shape=jax.ShapeDtypeStruct((_TOKENS, _F), jnp.float32),
        scratch_shapes=[
            pltpu.VMEM((_F, _F), jnp.bfloat16),
            pltpu.VMEM((_CB, _F), jnp.float32),
        ],
    )(x, W, b.reshape(_N_LAYERS, 1, _F))
